# baseline (speedup 1.0000x reference)
import functools

import jax
import jax.numpy as jnp
from jax import lax
from jax.experimental import pallas as pl
from jax.experimental.pallas import tpu as pltpu
from jax.experimental.pallas import tpu_sc as plsc

T, D, E, K, F = 2048, 2048, 8, 2, 1024
TT = 256
GT = 256
NT = 23
P = NT * GT
FS = 512
NW = 32
CH = 16


def _route_body(x_ref, rw_ref, pos_ref, w_ref, eot_ref, ntu_ref, loss_ref,
                rank1_s, rank2_s, oneh_s, psum_s, counts_s):
    i = pl.program_id(0)

    @pl.when(i == 0)
    def _init():
        counts_s[...] = jnp.zeros_like(counts_s)
        psum_s[...] = jnp.zeros_like(psum_s)

    @pl.when(i < T // TT)
    def _tile():
        xt = x_ref[...]
        logits = jnp.dot(xt.astype(jnp.bfloat16),
                         rw_ref[...].astype(jnp.bfloat16),
                         preferred_element_type=jnp.float32)
        m = jnp.max(logits, axis=-1, keepdims=True)
        p = jnp.exp(logits - m)
        probs = p / jnp.sum(p, axis=-1, keepdims=True)
        psum_s[...] += jnp.sum(probs, axis=0, keepdims=True)

        eidx = lax.broadcasted_iota(jnp.int32, (TT, E), 1)
        v1 = jnp.max(probs, axis=-1, keepdims=True)
        e1 = jnp.min(jnp.where(probs >= v1, eidx, E), axis=-1, keepdims=True)
        probs2 = jnp.where(eidx == e1, -1.0, probs)
        v2 = jnp.max(probs2, axis=-1, keepdims=True)
        e2 = jnp.min(jnp.where(probs2 >= v2, eidx, E), axis=-1, keepdims=True)

        oneh1 = (eidx == e1).astype(jnp.float32)
        oneh2 = (eidx == e2).astype(jnp.float32)
        oneh = oneh1 + oneh2

        r = lax.broadcasted_iota(jnp.int32, (TT, TT), 0)
        c = lax.broadcasted_iota(jnp.int32, (TT, TT), 1)
        tril = (c < r).astype(jnp.float32)
        excl = jnp.dot(tril, oneh, preferred_element_type=jnp.float32)
        cum = excl + counts_s[...]
        rk1 = jnp.sum(oneh1 * cum, axis=-1, keepdims=True)
        rk2 = jnp.sum(oneh2 * cum, axis=-1, keepdims=True)

        sl = pl.ds(i * TT, TT)
        rank1_s[sl, :] = rk1
        rank2_s[sl, :] = rk2
        oneh_s[sl, 0:E] = oneh1
        oneh_s[sl, E:2 * E] = oneh2
        ssum = v1 + v2
        w_ref[sl, 0:1] = v1 / ssum
        w_ref[sl, 1:2] = v2 / ssum
        counts_s[...] += jnp.sum(oneh, axis=0, keepdims=True)

    @pl.when(i == T // TT)
    def _final():
        counts = counts_s[...]
        padded = jnp.floor((counts + (GT - 1)) * (1.0 / GT)) * GT
        ec = lax.broadcasted_iota(jnp.int32, (E, E), 0)
        er = lax.broadcasted_iota(jnp.int32, (E, E), 1)
        tril8 = (ec < er).astype(jnp.float32)
        pstart = jnp.dot(padded, tril8, preferred_element_type=jnp.float32)

        o1 = oneh_s[:, 0:E]
        o2 = oneh_s[:, E:2 * E]
        p1 = jnp.sum(o1 * pstart, axis=-1, keepdims=True) + rank1_s[...]
        p2 = jnp.sum(o2 * pstart, axis=-1, keepdims=True) + rank2_s[...]
        pos_ref[:, 0:1] = p1.astype(jnp.int32)
        pos_ref[:, 1:2] = p2.astype(jnp.int32)

        ptend = (pstart + padded) * (1.0 / GT)
        jj = lax.broadcasted_iota(jnp.int32, (NT, E), 0).astype(jnp.float32)
        mask = (jnp.broadcast_to(ptend, (NT, E)) <= jj).astype(jnp.int32)
        eot_ref[...] = jnp.minimum(jnp.sum(mask, axis=-1, keepdims=True), E - 1)
        ntu_ref[...] = jnp.max(ptend, axis=-1, keepdims=True).astype(jnp.int32)

        loss_ref[...] = (E / (T * T)) * jnp.sum(
            counts * psum_s[...], axis=-1, keepdims=True)


def _route(x, router_w):
    return pl.pallas_call(
        _route_body,
        grid=(T // TT + 1,),
        in_specs=[
            pl.BlockSpec((TT, D), lambda i: (jnp.minimum(i, T // TT - 1), 0)),
            pl.BlockSpec((D, E), lambda i: (0, 0)),
        ],
        out_specs=[
            pl.BlockSpec((T, K), lambda i: (0, 0)),
            pl.BlockSpec((T, K), lambda i: (0, 0)),
            pl.BlockSpec((NT, 1), lambda i: (0, 0)),
            pl.BlockSpec((1, 1), lambda i: (0, 0)),
            pl.BlockSpec((1, 1), lambda i: (0, 0)),
        ],
        out_shape=[
            jax.ShapeDtypeStruct((T, K), jnp.int32),
            jax.ShapeDtypeStruct((T, K), jnp.float32),
            jax.ShapeDtypeStruct((NT, 1), jnp.int32),
            jax.ShapeDtypeStruct((1, 1), jnp.int32),
            jax.ShapeDtypeStruct((1, 1), jnp.float32),
        ],
        scratch_shapes=[
            pltpu.VMEM((T, 1), jnp.float32),
            pltpu.VMEM((T, 1), jnp.float32),
            pltpu.VMEM((T, 2 * E), jnp.float32),
            pltpu.VMEM((1, E), jnp.float32),
            pltpu.VMEM((1, E), jnp.float32),
        ],
    )(x, router_w)


def _sc_dispatch(x, pos0, pos1):
    mesh = plsc.VectorSubcoreMesh(core_axis_name="c", subcore_axis_name="s")
    per_w = T // NW

    @functools.partial(
        pl.kernel, mesh=mesh,
        out_type=jax.ShapeDtypeStruct((P, D), jnp.float32),
        scratch_types=[
            pltpu.VMEM((CH,), jnp.int32),
            pltpu.VMEM((CH, D), jnp.float32),
            pltpu.SemaphoreType.DMA,
        ],
    )
    def k(x_hbm, pos0_hbm, pos1_hbm, xs_hbm, idx_v, rows_v, sem):
        wid = lax.axis_index("s") * 2 + lax.axis_index("c")
        for j in range(per_w // CH):
            base = wid * per_w + j * CH
            pltpu.sync_copy(x_hbm.at[pl.ds(base, CH)], rows_v)
            pltpu.sync_copy(pos0_hbm.at[pl.ds(base, CH)], idx_v)
            pltpu.async_copy(rows_v, xs_hbm.at[idx_v], sem).wait()
            pltpu.sync_copy(pos1_hbm.at[pl.ds(base, CH)], idx_v)
            pltpu.async_copy(rows_v, xs_hbm.at[idx_v], sem).wait()

    return k(x, pos0, pos1)


def _gmm_body(eot_ref, ntu_ref, xs_ref, wg_ref, wu_ref, wd_ref, ys_ref):
    @pl.when(pl.program_id(0) < ntu_ref[0])
    def _():
        xt = xs_ref[...]
        g = jnp.dot(xt, wg_ref[0], preferred_element_type=jnp.float32)
        u = jnp.dot(xt, wu_ref[0], preferred_element_type=jnp.float32)
        h = (g * jax.nn.sigmoid(g)) * u
        ys_ref[...] = jnp.dot(h, wd_ref[0], preferred_element_type=jnp.float32)


def _gmm(eot, ntu, xs, w_gate, w_up, w_down):
    grid_spec = pltpu.PrefetchScalarGridSpec(
        num_scalar_prefetch=2,
        grid=(NT,),
        in_specs=[
            pl.BlockSpec((GT, D), lambda i, eot, ntu: (i, 0)),
            pl.BlockSpec((1, D, F), lambda i, eot, ntu: (eot[i], 0, 0)),
            pl.BlockSpec((1, D, F), lambda i, eot, ntu: (eot[i], 0, 0)),
            pl.BlockSpec((1, F, D), lambda i, eot, ntu: (eot[i], 0, 0)),
        ],
        out_specs=pl.BlockSpec((GT, D), lambda i, eot, ntu: (i, 0)),
    )
    return pl.pallas_call(
        _gmm_body,
        grid_spec=grid_spec,
        out_shape=jax.ShapeDtypeStruct((P, D), jnp.float32),
        compiler_params=pltpu.CompilerParams(
            dimension_semantics=("arbitrary",)),
    )(eot, ntu, xs, w_gate, w_up, w_down)


def _sc_combine(ys, pos_flat):
    mesh = plsc.VectorSubcoreMesh(core_axis_name="c", subcore_axis_name="s")
    per_w = (T * K) // NW

    @functools.partial(
        pl.kernel, mesh=mesh,
        out_type=jax.ShapeDtypeStruct((T * K, D), jnp.float32),
        scratch_types=[
            pltpu.VMEM((CH,), jnp.int32),
            pltpu.VMEM((CH, D), jnp.float32),
            pltpu.SemaphoreType.DMA,
        ],
    )
    def k(ys_hbm, pos_hbm, yc_hbm, idx_v, rows_v, sem):
        wid = lax.axis_index("s") * 2 + lax.axis_index("c")
        for j in range(per_w // CH):
            base = wid * per_w + j * CH
            pltpu.sync_copy(pos_hbm.at[pl.ds(base, CH)], idx_v)
            pltpu.async_copy(ys_hbm.at[idx_v], rows_v, sem).wait()
            pltpu.sync_copy(rows_v, yc_hbm.at[pl.ds(base, CH)])

    return k(ys, pos_flat)


def _shared_body(x_ref, wsg_ref, wsu_ref, wsd_ref, wgs_ref, out_ref):
    xt = x_ref[...]
    g = jnp.dot(xt, wsg_ref[...], preferred_element_type=jnp.float32)
    u = jnp.dot(xt, wsu_ref[...], preferred_element_type=jnp.float32)
    h = (g * jax.nn.sigmoid(g)) * u
    part = jnp.dot(h, wsd_ref[...], preferred_element_type=jnp.float32)
    gate = jax.nn.sigmoid(jnp.dot(xt, wgs_ref[...],
                                  preferred_element_type=jnp.float32))
    out_ref[...] = part * gate


def _shared(x, ws_gate, ws_up, ws_down, wg_shared, half):
    TH = T // 2
    return pl.pallas_call(
        _shared_body,
        grid=(TH // TT,),
        in_specs=[
            pl.BlockSpec((TT, D), lambda i: (half * (TH // TT) + i, 0)),
            pl.BlockSpec((D, F), lambda i: (0, 0)),
            pl.BlockSpec((D, F), lambda i: (0, 0)),
            pl.BlockSpec((F, D), lambda i: (0, 0)),
            pl.BlockSpec((D, 1), lambda i: (0, 0)),
        ],
        out_specs=pl.BlockSpec((TT, D), lambda i: (i, 0)),
        out_shape=jax.ShapeDtypeStruct((TH, D), jnp.float32),
        compiler_params=pltpu.CompilerParams(
            dimension_semantics=("arbitrary",)),
    )(x, ws_gate, ws_up, ws_down, wg_shared)


def _final_body(yc_ref, w_ref, sha_ref, shb_ref, out_ref):
    i = pl.program_id(0)
    yc = yc_ref[...].reshape(TT, K, D)
    y0 = yc[:, 0, :]
    y1 = yc[:, 1, :]
    w0 = w_ref[:, 0:1]
    w1 = w_ref[:, 1:2]
    base = w0 * y0 + w1 * y1
    nh = T // TT // 2

    @pl.when(i < nh)
    def _():
        out_ref[...] = base + sha_ref[...]

    @pl.when(i >= nh)
    def _():
        out_ref[...] = base + shb_ref[...]


def _final(yc, w, sha, shb):
    nh = T // TT // 2
    return pl.pallas_call(
        _final_body,
        grid=(T // TT,),
        in_specs=[
            pl.BlockSpec((K * TT, D), lambda i: (i, 0)),
            pl.BlockSpec((TT, K), lambda i: (i, 0)),
            pl.BlockSpec((TT, D), lambda i: (jnp.minimum(i, nh - 1), 0)),
            pl.BlockSpec((TT, D), lambda i: (jnp.maximum(i - nh, 0), 0)),
        ],
        out_specs=pl.BlockSpec((TT, D), lambda i: (i, 0)),
        out_shape=jax.ShapeDtypeStruct((T, D), jnp.float32),
        compiler_params=pltpu.CompilerParams(
            dimension_semantics=("arbitrary",)),
    )(yc, w, sha, shb)


def kernel(hidden_states, router_w, w_gate, w_up, w_down,
           ws_gate, ws_up, ws_down, wg_shared, deterministic=True):
    b, s, d = hidden_states.shape
    x = hidden_states.reshape(-1, d)

    pos, w, eot, ntu, loss = _route(x, router_w)
    xs = _sc_dispatch(x, pos[:, 0], pos[:, 1])
    ys = _gmm(eot.reshape(NT), ntu.reshape(1), xs, w_gate, w_up, w_down)
    yc = _sc_combine(ys, pos.reshape(-1))
    sha = _shared(x, ws_gate, ws_up, ws_down, wg_shared, 0)
    shb = _shared(x, ws_gate, ws_up, ws_down, wg_shared, 1)
    out = _final(yc, w, sha, shb)

    return out.reshape(b, s, d), loss[0, 0]


# speedup vs baseline: 1.0522x; 1.0522x over previous
import functools

import jax
import jax.numpy as jnp
from jax import lax
from jax.experimental import pallas as pl
from jax.experimental.pallas import tpu as pltpu
from jax.experimental.pallas import tpu_sc as plsc

T, D, E, K, F = 2048, 2048, 8, 2, 1024
TT = 256
GT = 256
NT = 23
P = NT * GT
FS = 512
NW = 32
CH = 32


def _route_body(x_ref, rw_ref, pos_ref, w_ref, eot_ref, ntu_ref, loss_ref,
                rank1_s, rank2_s, oneh_s, psum_s, counts_s):
    i = pl.program_id(0)

    @pl.when(i == 0)
    def _init():
        counts_s[...] = jnp.zeros_like(counts_s)
        psum_s[...] = jnp.zeros_like(psum_s)

    @pl.when(i < T // TT)
    def _tile():
        xt = x_ref[...]
        logits = jnp.dot(xt.astype(jnp.bfloat16),
                         rw_ref[...].astype(jnp.bfloat16),
                         preferred_element_type=jnp.float32)
        m = jnp.max(logits, axis=-1, keepdims=True)
        p = jnp.exp(logits - m)
        probs = p / jnp.sum(p, axis=-1, keepdims=True)
        psum_s[...] += jnp.sum(probs, axis=0, keepdims=True)

        eidx = lax.broadcasted_iota(jnp.int32, (TT, E), 1)
        v1 = jnp.max(probs, axis=-1, keepdims=True)
        e1 = jnp.min(jnp.where(probs >= v1, eidx, E), axis=-1, keepdims=True)
        probs2 = jnp.where(eidx == e1, -1.0, probs)
        v2 = jnp.max(probs2, axis=-1, keepdims=True)
        e2 = jnp.min(jnp.where(probs2 >= v2, eidx, E), axis=-1, keepdims=True)

        oneh1 = (eidx == e1).astype(jnp.float32)
        oneh2 = (eidx == e2).astype(jnp.float32)
        oneh = oneh1 + oneh2

        r = lax.broadcasted_iota(jnp.int32, (TT, TT), 0)
        c = lax.broadcasted_iota(jnp.int32, (TT, TT), 1)
        tril = (c < r).astype(jnp.float32)
        excl = jnp.dot(tril, oneh, preferred_element_type=jnp.float32)
        cum = excl + counts_s[...]
        rk1 = jnp.sum(oneh1 * cum, axis=-1, keepdims=True)
        rk2 = jnp.sum(oneh2 * cum, axis=-1, keepdims=True)

        sl = pl.ds(i * TT, TT)
        rank1_s[sl, :] = rk1
        rank2_s[sl, :] = rk2
        oneh_s[sl, 0:E] = oneh1
        oneh_s[sl, E:2 * E] = oneh2
        ssum = v1 + v2
        w_ref[sl, 0:1] = v1 / ssum
        w_ref[sl, 1:2] = v2 / ssum
        counts_s[...] += jnp.sum(oneh, axis=0, keepdims=True)

    @pl.when(i == T // TT)
    def _final():
        counts = counts_s[...]
        padded = jnp.floor((counts + (GT - 1)) * (1.0 / GT)) * GT
        ec = lax.broadcasted_iota(jnp.int32, (E, E), 0)
        er = lax.broadcasted_iota(jnp.int32, (E, E), 1)
        tril8 = (ec < er).astype(jnp.float32)
        pstart = jnp.dot(padded, tril8, preferred_element_type=jnp.float32)

        o1 = oneh_s[:, 0:E]
        o2 = oneh_s[:, E:2 * E]
        p1 = jnp.sum(o1 * pstart, axis=-1, keepdims=True) + rank1_s[...]
        p2 = jnp.sum(o2 * pstart, axis=-1, keepdims=True) + rank2_s[...]
        pos_ref[:, 0:1] = p1.astype(jnp.int32)
        pos_ref[:, 1:2] = p2.astype(jnp.int32)

        ptend = (pstart + padded) * (1.0 / GT)
        jj = lax.broadcasted_iota(jnp.int32, (NT, E), 0).astype(jnp.float32)
        mask = (jnp.broadcast_to(ptend, (NT, E)) <= jj).astype(jnp.int32)
        eot_ref[...] = jnp.minimum(jnp.sum(mask, axis=-1, keepdims=True), E - 1)
        ntu_ref[...] = jnp.max(ptend, axis=-1, keepdims=True).astype(jnp.int32)

        loss_ref[...] = (E / (T * T)) * jnp.sum(
            counts * psum_s[...], axis=-1, keepdims=True)


def _route(x, router_w):
    return pl.pallas_call(
        _route_body,
        grid=(T // TT + 1,),
        in_specs=[
            pl.BlockSpec((TT, D), lambda i: (jnp.minimum(i, T // TT - 1), 0)),
            pl.BlockSpec((D, E), lambda i: (0, 0)),
        ],
        out_specs=[
            pl.BlockSpec((T, K), lambda i: (0, 0)),
            pl.BlockSpec((T, K), lambda i: (0, 0)),
            pl.BlockSpec((NT, 1), lambda i: (0, 0)),
            pl.BlockSpec((1, 1), lambda i: (0, 0)),
            pl.BlockSpec((1, 1), lambda i: (0, 0)),
        ],
        out_shape=[
            jax.ShapeDtypeStruct((T, K), jnp.int32),
            jax.ShapeDtypeStruct((T, K), jnp.float32),
            jax.ShapeDtypeStruct((NT, 1), jnp.int32),
            jax.ShapeDtypeStruct((1, 1), jnp.int32),
            jax.ShapeDtypeStruct((1, 1), jnp.float32),
        ],
        scratch_shapes=[
            pltpu.VMEM((T, 1), jnp.float32),
            pltpu.VMEM((T, 1), jnp.float32),
            pltpu.VMEM((T, 2 * E), jnp.float32),
            pltpu.VMEM((1, E), jnp.float32),
            pltpu.VMEM((1, E), jnp.float32),
        ],
    )(x, router_w)


def _sc_dispatch(x, pos0, pos1):
    mesh = plsc.VectorSubcoreMesh(core_axis_name="c", subcore_axis_name="s")
    per_w = T // NW

    @functools.partial(
        pl.kernel, mesh=mesh,
        out_type=jax.ShapeDtypeStruct((P, D), jnp.float32),
        scratch_types=[
            pltpu.VMEM((CH,), jnp.int32),
            pltpu.VMEM((CH, D), jnp.float32),
            pltpu.SemaphoreType.DMA,
        ],
    )
    def k(x_hbm, pos0_hbm, pos1_hbm, xs_hbm, idx_v, rows_v, sem):
        wid = lax.axis_index("s") * 2 + lax.axis_index("c")
        for j in range(per_w // CH):
            base = wid * per_w + j * CH
            pltpu.sync_copy(x_hbm.at[pl.ds(base, CH)], rows_v)
            pltpu.sync_copy(pos0_hbm.at[pl.ds(base, CH)], idx_v)
            pltpu.async_copy(rows_v, xs_hbm.at[idx_v], sem).wait()
            pltpu.sync_copy(pos1_hbm.at[pl.ds(base, CH)], idx_v)
            pltpu.async_copy(rows_v, xs_hbm.at[idx_v], sem).wait()

    return k(x, pos0, pos1)


def _gmm_body(eot_ref, ntu_ref, xs_ref, wg_ref, wu_ref, wd_ref, ys_ref):
    @pl.when(pl.program_id(0) < ntu_ref[0])
    def _():
        xt = xs_ref[...]
        g = jnp.dot(xt, wg_ref[0], preferred_element_type=jnp.float32)
        u = jnp.dot(xt, wu_ref[0], preferred_element_type=jnp.float32)
        h = (g * jax.nn.sigmoid(g)) * u
        ys_ref[...] = jnp.dot(h, wd_ref[0], preferred_element_type=jnp.float32)


def _gmm(eot, ntu, xs, w_gate, w_up, w_down):
    grid_spec = pltpu.PrefetchScalarGridSpec(
        num_scalar_prefetch=2,
        grid=(NT,),
        in_specs=[
            pl.BlockSpec((GT, D), lambda i, eot, ntu: (i, 0)),
            pl.BlockSpec((1, D, F), lambda i, eot, ntu: (eot[i], 0, 0)),
            pl.BlockSpec((1, D, F), lambda i, eot, ntu: (eot[i], 0, 0)),
            pl.BlockSpec((1, F, D), lambda i, eot, ntu: (eot[i], 0, 0)),
        ],
        out_specs=pl.BlockSpec((GT, D), lambda i, eot, ntu: (i, 0)),
    )
    return pl.pallas_call(
        _gmm_body,
        grid_spec=grid_spec,
        out_shape=jax.ShapeDtypeStruct((P, D), jnp.float32),
        compiler_params=pltpu.CompilerParams(
            dimension_semantics=("arbitrary",)),
    )(eot, ntu, xs, w_gate, w_up, w_down)


def _sc_combine(ys, pos_flat):
    mesh = plsc.VectorSubcoreMesh(core_axis_name="c", subcore_axis_name="s")
    per_w = (T * K) // NW

    @functools.partial(
        pl.kernel, mesh=mesh,
        out_type=jax.ShapeDtypeStruct((T * K, D), jnp.float32),
        scratch_types=[
            pltpu.VMEM((CH,), jnp.int32),
            pltpu.VMEM((CH, D), jnp.float32),
            pltpu.SemaphoreType.DMA,
        ],
    )
    def k(ys_hbm, pos_hbm, yc_hbm, idx_v, rows_v, sem):
        wid = lax.axis_index("s") * 2 + lax.axis_index("c")
        for j in range(per_w // CH):
            base = wid * per_w + j * CH
            pltpu.sync_copy(pos_hbm.at[pl.ds(base, CH)], idx_v)
            pltpu.async_copy(ys_hbm.at[idx_v], rows_v, sem).wait()
            pltpu.sync_copy(rows_v, yc_hbm.at[pl.ds(base, CH)])

    return k(ys, pos_flat)


def _shared_body(x_ref, wsg_ref, wsu_ref, wsd_ref, wgs_ref, out_ref):
    xt = x_ref[...]
    g = jnp.dot(xt, wsg_ref[...], preferred_element_type=jnp.float32)
    u = jnp.dot(xt, wsu_ref[...], preferred_element_type=jnp.float32)
    h = (g * jax.nn.sigmoid(g)) * u
    part = jnp.dot(h, wsd_ref[...], preferred_element_type=jnp.float32)
    gate = jax.nn.sigmoid(jnp.dot(xt, wgs_ref[...],
                                  preferred_element_type=jnp.float32))
    out_ref[...] = part * gate


def _shared(x, ws_gate, ws_up, ws_down, wg_shared):
    return pl.pallas_call(
        _shared_body,
        grid=(T // TT,),
        in_specs=[
            pl.BlockSpec((TT, D), lambda i: (i, 0)),
            pl.BlockSpec((D, F), lambda i: (0, 0)),
            pl.BlockSpec((D, F), lambda i: (0, 0)),
            pl.BlockSpec((F, D), lambda i: (0, 0)),
            pl.BlockSpec((D, 1), lambda i: (0, 0)),
        ],
        out_specs=pl.BlockSpec((TT, D), lambda i: (i, 0)),
        out_shape=jax.ShapeDtypeStruct((T, D), jnp.float32),
        compiler_params=pltpu.CompilerParams(
            dimension_semantics=("arbitrary",)),
    )(x, ws_gate, ws_up, ws_down, wg_shared)


def _final_body(yc_ref, w_ref, sh_ref, out_ref):
    yc = yc_ref[...].reshape(TT, K, D)
    y0 = yc[:, 0, :]
    y1 = yc[:, 1, :]
    w0 = w_ref[:, 0:1]
    w1 = w_ref[:, 1:2]
    out_ref[...] = w0 * y0 + w1 * y1 + sh_ref[...]


def _final(yc, w, sh):
    return pl.pallas_call(
        _final_body,
        grid=(T // TT,),
        in_specs=[
            pl.BlockSpec((K * TT, D), lambda i: (i, 0)),
            pl.BlockSpec((TT, K), lambda i: (i, 0)),
            pl.BlockSpec((TT, D), lambda i: (i, 0)),
        ],
        out_specs=pl.BlockSpec((TT, D), lambda i: (i, 0)),
        out_shape=jax.ShapeDtypeStruct((T, D), jnp.float32),
        compiler_params=pltpu.CompilerParams(
            dimension_semantics=("arbitrary",)),
    )(yc, w, sh)


def kernel(hidden_states, router_w, w_gate, w_up, w_down,
           ws_gate, ws_up, ws_down, wg_shared, deterministic=True):
    b, s, d = hidden_states.shape
    x = hidden_states.reshape(-1, d)

    pos, w, eot, ntu, loss = _route(x, router_w)
    xs = _sc_dispatch(x, pos[:, 0], pos[:, 1])
    ys = _gmm(eot.reshape(NT), ntu.reshape(1), xs, w_gate, w_up, w_down)
    yc = _sc_combine(ys, pos.reshape(-1))
    sh = _shared(x, ws_gate, ws_up, ws_down, wg_shared)
    out = _final(yc, w, sh)

    return out.reshape(b, s, d), loss[0, 0]
